# baseline (speedup 1.0000x reference)
import functools

import jax
import jax.numpy as jnp
from jax import lax
from jax.experimental import pallas as pl
from jax.experimental.pallas import tpu as pltpu
from jax.experimental.pallas import tpu_sc as plsc

BATCH = 16384
DIM = 64
ROWS = 1500000
LANE = 128
KTILE = 8
NUM_CORES = 2
NUM_SUBCORES = 16
NUM_WORKERS = NUM_CORES * NUM_SUBCORES
B_PER_W = BATCH // NUM_WORKERS
SUB = 4

_mesh = plsc.VectorSubcoreMesh(core_axis_name="c", subcore_axis_name="s")


@functools.partial(
    pl.kernel,
    mesh=_mesh,
    out_type=jax.ShapeDtypeStruct((DIM, BATCH), jnp.float32),
    scratch_types=[
        pltpu.VMEM((B_PER_W,), jnp.int32),
        pltpu.VMEM((SUB * DIM, LANE), jnp.float32),
        pltpu.VMEM((SUB * DIM, LANE), jnp.float32),
        pltpu.VMEM((DIM, B_PER_W), jnp.float32),
        pltpu.SemaphoreType.DMA,
        pltpu.SemaphoreType.DMA,
    ],
    compiler_params=pltpu.CompilerParams(needs_layout_passes=False),
)
def _sc_gather(table_hbm, idx_hbm, out_hbm, idx_v, bufa, bufb, slab,
               sema, semb):
    wid = lax.axis_index("s") * NUM_CORES + lax.axis_index("c")
    base = wid * B_PER_W
    pltpu.sync_copy(idx_hbm.at[pl.ds(base, B_PER_W)], idx_v)

    lane = lax.iota(jnp.int32, 16)
    bufs = (bufa, bufb)
    sems = (sema, semb)

    def fire(cvec, i):
        buf, sem = bufs[i % 2], sems[i % 2]
        buf4 = buf.reshape(SUB, KTILE, 8, LANE)
        copies = []
        for t in range(SUB):
            c = pl.multiple_of(cvec[SUB * i + t] * LANE, LANE)
            copies.append(pltpu.async_copy(
                table_hbm.at[pl.ds(0, KTILE), pl.ds(0, 8), pl.ds(c, LANE)],
                buf4.at[t],
                sem,
            ))
        return copies

    def select(g, jvec, i, copies):
        buf = bufs[i % 2]
        for cp in copies:
            cp.wait()
        for t in range(SUB):
            e = g * 16 + SUB * i + t
            col = lax.broadcast(jvec[SUB * i + t], (16,))
            for kq in range(DIM // 16):
                rows = t * DIM + kq * 16 + lane
                vals = plsc.load_gather(buf, [rows, col])
                plsc.store_scatter(slab, [kq * 16 + lane,
                                          lax.broadcast(e, (16,))], vals)

    def body(g, carry):
        ids = idx_v[pl.ds(g * 16, 16)]
        cvec = ids >> 7
        jvec = ids & (LANE - 1)
        c0 = fire(cvec, 0)
        c1 = fire(cvec, 1)
        select(g, jvec, 0, c0)
        c2 = fire(cvec, 2)
        select(g, jvec, 1, c1)
        c3 = fire(cvec, 3)
        select(g, jvec, 2, c2)
        select(g, jvec, 3, c3)
        return carry

    lax.fori_loop(0, B_PER_W // 16, body, 0)
    pltpu.sync_copy(slab, out_hbm.at[pl.ds(0, DIM), pl.ds(base, B_PER_W)])


def kernel(embedding_weight, batch):
    table4 = embedding_weight.T.reshape(KTILE, 8, ROWS)
    idx = batch.astype(jnp.int32)
    srt = jnp.argsort(idx)
    out_t = _sc_gather(table4, idx[srt])
    out2 = out_t.T
    return jnp.zeros((BATCH, DIM), jnp.float32).at[srt].set(out2)


# speedup vs baseline: 1.5723x; 1.5723x over previous
import functools

import jax
import jax.numpy as jnp
from jax import lax
from jax.experimental import pallas as pl
from jax.experimental.pallas import tpu as pltpu
from jax.experimental.pallas import tpu_sc as plsc

BATCH = 16384
DIM = 64
ROWS = 1500000
LANE = 128
KTILE = 8
NUM_CORES = 2
NUM_SUBCORES = 16
NUM_WORKERS = NUM_CORES * NUM_SUBCORES
B_PER_W = BATCH // NUM_WORKERS
SUB = 4
NSLOT = 8

_mesh = plsc.VectorSubcoreMesh(core_axis_name="c", subcore_axis_name="s")


@functools.partial(
    pl.kernel,
    mesh=_mesh,
    out_type=jax.ShapeDtypeStruct((DIM, BATCH), jnp.float32),
    scratch_types=[
        pltpu.VMEM((B_PER_W,), jnp.int32),
        pltpu.VMEM((NSLOT * DIM, LANE), jnp.float32),
        pltpu.VMEM((DIM, B_PER_W), jnp.float32),
        pltpu.SemaphoreType.DMA,
        pltpu.SemaphoreType.DMA,
    ],
    compiler_params=pltpu.CompilerParams(needs_layout_passes=False),
)
def _sc_gather(table_hbm, idx_hbm, out_hbm, idx_v, buf, slab, sema, semb):
    wid = lax.axis_index("s") * NUM_CORES + lax.axis_index("c")
    base = wid * B_PER_W
    pltpu.sync_copy(idx_hbm.at[pl.ds(base, B_PER_W)], idx_v)

    lane = lax.iota(jnp.int32, 16)
    sems = (sema, semb)
    buf4 = buf.reshape(NSLOT, KTILE, 8, LANE)

    def fire(cvec, i, st):
        prev_c, fire_cnt, prev_slot = st
        sem = sems[i % 2]
        meta = []
        for t in range(SUB):
            c = cvec[SUB * i + t]
            pred = c != prev_c
            slot = lax.select(pred, fire_cnt & (NSLOT - 1), prev_slot)

            @pl.when(pred)
            def _():
                pltpu.async_copy(
                    table_hbm.at[pl.ds(0, KTILE), pl.ds(0, 8),
                                 pl.ds(pl.multiple_of(c * LANE, LANE), LANE)],
                    buf4.at[slot],
                    sem,
                )

            meta.append((pred, slot))
            fire_cnt = fire_cnt + pred.astype(jnp.int32)
            prev_c, prev_slot = c, slot
        return meta, (prev_c, fire_cnt, prev_slot)

    def select(g, jvec, i, meta):
        sem = sems[i % 2]
        for t in range(SUB):
            pred, slot = meta[t]

            @pl.when(pred)
            def _():
                pltpu.make_async_copy(
                    table_hbm.at[pl.ds(0, KTILE), pl.ds(0, 8), pl.ds(0, LANE)],
                    buf4.at[slot], sem,
                ).wait()

            e = g * 16 + SUB * i + t
            col = lax.broadcast(jvec[SUB * i + t], (16,))
            for kq in range(DIM // 16):
                rows = slot * DIM + kq * 16 + lane
                vals = plsc.load_gather(buf, [rows, col])
                plsc.store_scatter(slab, [kq * 16 + lane,
                                          lax.broadcast(e, (16,))], vals)

    def body(g, st):
        ids = idx_v[pl.ds(g * 16, 16)]
        cvec = ids >> 7
        jvec = ids & (LANE - 1)
        m0, st = fire(cvec, 0, st)
        m1, st = fire(cvec, 1, st)
        select(g, jvec, 0, m0)
        m2, st = fire(cvec, 2, st)
        select(g, jvec, 1, m1)
        m3, st = fire(cvec, 3, st)
        select(g, jvec, 2, m2)
        select(g, jvec, 3, m3)
        return st

    init = (jnp.int32(-1), jnp.int32(0), jnp.int32(0))
    lax.fori_loop(0, B_PER_W // 16, body, init)
    pltpu.sync_copy(slab, out_hbm.at[pl.ds(0, DIM), pl.ds(base, B_PER_W)])


def kernel(embedding_weight, batch):
    table4 = embedding_weight.T.reshape(KTILE, 8, ROWS)
    idx = batch.astype(jnp.int32)
    srt = jnp.argsort(idx)
    out_t = _sc_gather(table4, idx[srt])
    out2 = out_t.T
    return jnp.zeros((BATCH, DIM), jnp.float32).at[srt].set(out2)


# speedup vs baseline: 1.8695x; 1.1890x over previous
import functools

import jax
import jax.numpy as jnp
from jax import lax
from jax.experimental import pallas as pl
from jax.experimental.pallas import tpu as pltpu
from jax.experimental.pallas import tpu_sc as plsc

BATCH = 16384
DIM = 64
ROWS = 1500000
LANE = 128
KTILE = 8
NUM_CORES = 2
NUM_SUBCORES = 16
NUM_WORKERS = NUM_CORES * NUM_SUBCORES
B_PER_W = BATCH // NUM_WORKERS
SUB = 4
NSLOT = 8

_mesh = plsc.VectorSubcoreMesh(core_axis_name="c", subcore_axis_name="s")


@functools.partial(
    pl.kernel,
    mesh=_mesh,
    out_type=jax.ShapeDtypeStruct((DIM, BATCH), jnp.float32),
    scratch_types=[
        pltpu.VMEM((B_PER_W,), jnp.int32),
        pltpu.VMEM((NSLOT * DIM, LANE), jnp.float32),
        pltpu.VMEM((DIM, B_PER_W), jnp.float32),
        pltpu.SemaphoreType.DMA,
        pltpu.SemaphoreType.DMA,
    ],
    compiler_params=pltpu.CompilerParams(needs_layout_passes=False),
)
def _sc_gather(table_hbm, idx_hbm, out_hbm, idx_v, buf, slab, sema, semb):
    wid = lax.axis_index("s") * NUM_CORES + lax.axis_index("c")
    base = wid * B_PER_W
    pltpu.sync_copy(idx_hbm.at[pl.ds(base, B_PER_W)], idx_v)

    lane = lax.iota(jnp.int32, 16)
    sems = (sema, semb)
    buf4 = buf.reshape(NSLOT, KTILE, 8, LANE)

    def fire(cvec, i, st):
        prev_c, fire_cnt, prev_slot = st
        sem = sems[i % 2]
        meta = []
        for t in range(SUB):
            c = cvec[SUB * i + t]
            pred = c != prev_c
            slot = lax.select(pred, fire_cnt & (NSLOT - 1), prev_slot)

            @pl.when(pred)
            def _():
                pltpu.async_copy(
                    table_hbm.at[pl.ds(0, KTILE), pl.ds(0, 8),
                                 pl.ds(pl.multiple_of(c * LANE, LANE), LANE)],
                    buf4.at[slot],
                    sem,
                )

            meta.append((pred, slot))
            fire_cnt = fire_cnt + pred.astype(jnp.int32)
            prev_c, prev_slot = c, slot
        return meta, (prev_c, fire_cnt, prev_slot)

    def select(g, jvec, i, meta):
        sem = sems[i % 2]
        for t in range(SUB):
            pred, slot = meta[t]

            @pl.when(pred)
            def _():
                pltpu.make_async_copy(
                    table_hbm.at[pl.ds(0, KTILE), pl.ds(0, 8), pl.ds(0, LANE)],
                    buf4.at[slot], sem,
                ).wait()

            e = g * 16 + SUB * i + t
            col = lax.broadcast(jvec[SUB * i + t], (16,))
            for kq in range(DIM // 16):
                rows = slot * DIM + kq * 16 + lane
                vals = plsc.load_gather(buf, [rows, col])
                plsc.store_scatter(slab, [kq * 16 + lane,
                                          lax.broadcast(e, (16,))], vals)

    def body(g, st):
        ids = idx_v[pl.ds(g * 16, 16)]
        cvec = ids >> 7
        jvec = ids & (LANE - 1)
        m0, st = fire(cvec, 0, st)
        m1, st = fire(cvec, 1, st)
        select(g, jvec, 0, m0)
        m2, st = fire(cvec, 2, st)
        select(g, jvec, 1, m1)
        m3, st = fire(cvec, 3, st)
        select(g, jvec, 2, m2)
        select(g, jvec, 3, m3)
        return st

    init = (jnp.int32(-1), jnp.int32(0), jnp.int32(0))
    lax.fori_loop(0, B_PER_W // 16, body, init)
    pltpu.sync_copy(slab, out_hbm.at[pl.ds(0, DIM), pl.ds(base, B_PER_W)])


def kernel(embedding_weight, batch):
    table4 = embedding_weight.T.reshape(KTILE, 8, ROWS)
    idx = batch.astype(jnp.int32)
    srt = jnp.argsort(idx)
    out_t = _sc_gather(table4, idx[srt])
    out2 = out_t.T
    inv = jnp.zeros((BATCH,), jnp.int32).at[srt].set(
        jnp.arange(BATCH, dtype=jnp.int32))
    return out2[inv]


# speedup vs baseline: 1.9532x; 1.0448x over previous
import functools

import jax
import jax.numpy as jnp
from jax import lax
from jax.experimental import pallas as pl
from jax.experimental.pallas import tpu as pltpu
from jax.experimental.pallas import tpu_sc as plsc

BATCH = 16384
DIM = 64
ROWS = 1500000
LANE = 128
KTILE = 8
NUM_CORES = 2
NUM_SUBCORES = 16
NUM_WORKERS = NUM_CORES * NUM_SUBCORES
B_PER_W = BATCH // NUM_WORKERS
SUB = 4
NSLOT = 8

_mesh = plsc.VectorSubcoreMesh(core_axis_name="c", subcore_axis_name="s")


@functools.partial(
    pl.kernel,
    mesh=_mesh,
    out_type=jax.ShapeDtypeStruct((DIM, BATCH), jnp.float32),
    scratch_types=[
        pltpu.VMEM((B_PER_W,), jnp.int32),
        pltpu.VMEM((NSLOT * DIM, LANE), jnp.float32),
        pltpu.VMEM((DIM, B_PER_W), jnp.float32),
        pltpu.SemaphoreType.DMA,
        pltpu.SemaphoreType.DMA,
    ],
    compiler_params=pltpu.CompilerParams(needs_layout_passes=False),
)
def _sc_gather(table_hbm, idx_hbm, out_hbm, idx_v, buf, slab, sema, semb):
    wid = lax.axis_index("s") * NUM_CORES + lax.axis_index("c")
    base = wid * B_PER_W
    pltpu.sync_copy(idx_hbm.at[pl.ds(base, B_PER_W)], idx_v)

    lane = lax.iota(jnp.int32, 16)
    sems = (sema, semb)
    buf4 = buf.reshape(NSLOT, KTILE, 8, LANE)

    def fire(cvec, i, st):
        prev_c, fire_cnt, prev_slot = st
        sem = sems[i % 2]
        meta = []
        for t in range(SUB):
            c = cvec[SUB * i + t]
            pred = c != prev_c
            slot = lax.select(pred, fire_cnt & (NSLOT - 1), prev_slot)

            @pl.when(pred)
            def _():
                pltpu.async_copy(
                    table_hbm.at[pl.ds(0, KTILE), pl.ds(0, 8),
                                 pl.ds(pl.multiple_of(c * LANE, LANE), LANE)],
                    buf4.at[slot],
                    sem,
                )

            meta.append((pred, slot))
            fire_cnt = fire_cnt + pred.astype(jnp.int32)
            prev_c, prev_slot = c, slot
        return meta, (prev_c, fire_cnt, prev_slot)

    def select(g, jvec, i, meta):
        sem = sems[i % 2]
        for t in range(SUB):
            pred, slot = meta[t]

            @pl.when(pred)
            def _():
                pltpu.make_async_copy(
                    table_hbm.at[pl.ds(0, KTILE), pl.ds(0, 8), pl.ds(0, LANE)],
                    buf4.at[slot], sem,
                ).wait()

            e = g * 16 + SUB * i + t
            col = lax.broadcast(jvec[SUB * i + t], (16,))
            for kq in range(DIM // 16):
                rows = slot * DIM + kq * 16 + lane
                vals = plsc.load_gather(buf, [rows, col])
                plsc.store_scatter(slab, [kq * 16 + lane,
                                          lax.broadcast(e, (16,))], vals)

    def body(g, st):
        ids = idx_v[pl.ds(g * 16, 16)]
        cvec = ids >> 7
        jvec = ids & (LANE - 1)
        m0, st = fire(cvec, 0, st)
        m1, st = fire(cvec, 1, st)
        select(g, jvec, 0, m0)
        m2, st = fire(cvec, 2, st)
        select(g, jvec, 1, m1)
        m3, st = fire(cvec, 3, st)
        select(g, jvec, 2, m2)
        select(g, jvec, 3, m3)
        return st

    init = (jnp.int32(-1), jnp.int32(0), jnp.int32(0))
    lax.fori_loop(0, B_PER_W // 16, body, init)
    pltpu.sync_copy(slab, out_hbm.at[pl.ds(0, DIM), pl.ds(base, B_PER_W)])


def kernel(embedding_weight, batch):
    table4 = embedding_weight.T.reshape(KTILE, 8, ROWS)
    idx = batch.astype(jnp.int32)
    ids_sorted, srt = lax.sort(
        [idx, jnp.arange(BATCH, dtype=jnp.int32)], num_keys=1)
    out_t = _sc_gather(table4, ids_sorted)
    out2 = out_t.T
    inv = jnp.zeros((BATCH,), jnp.int32).at[srt].set(
        jnp.arange(BATCH, dtype=jnp.int32))
    return out2[inv]


# speedup vs baseline: 2.0316x; 1.0401x over previous
import functools

import jax
import jax.numpy as jnp
from jax import lax
from jax.experimental import pallas as pl
from jax.experimental.pallas import tpu as pltpu
from jax.experimental.pallas import tpu_sc as plsc

BATCH = 16384
DIM = 64
ROWS = 1500000
LANE = 128
KTILE = 8
NUM_CORES = 2
NUM_SUBCORES = 16
NUM_WORKERS = NUM_CORES * NUM_SUBCORES
B_PER_W = BATCH // NUM_WORKERS
SUB = 4
NSLOT = 8

_mesh = plsc.VectorSubcoreMesh(core_axis_name="c", subcore_axis_name="s")


@functools.partial(
    pl.kernel,
    mesh=_mesh,
    out_type=jax.ShapeDtypeStruct((BATCH, DIM), jnp.float32),
    scratch_types=[
        pltpu.VMEM((B_PER_W,), jnp.int32),
        pltpu.VMEM((NSLOT * DIM, LANE), jnp.float32),
        pltpu.VMEM((B_PER_W // 2, DIM), jnp.float32),
        pltpu.SemaphoreType.DMA,
        pltpu.SemaphoreType.DMA,
    ],
    compiler_params=pltpu.CompilerParams(needs_layout_passes=False),
)
def _sc_gather(table_hbm, idx_hbm, out_hbm, idx_v, buf, slab, sema, semb):
    wid = lax.axis_index("s") * NUM_CORES + lax.axis_index("c")
    base = wid * B_PER_W
    pltpu.sync_copy(idx_hbm.at[pl.ds(base, B_PER_W)], idx_v)

    lane = lax.iota(jnp.int32, 16)
    sems = (sema, semb)
    buf4 = buf.reshape(NSLOT, KTILE, 8, LANE)

    def fire(cvec, i, st):
        prev_c, fire_cnt, prev_slot = st
        sem = sems[i % 2]
        meta = []
        for t in range(SUB):
            c = cvec[SUB * i + t]
            pred = c != prev_c
            slot = lax.select(pred, fire_cnt & (NSLOT - 1), prev_slot)

            @pl.when(pred)
            def _():
                pltpu.async_copy(
                    table_hbm.at[pl.ds(0, KTILE), pl.ds(0, 8),
                                 pl.ds(pl.multiple_of(c * LANE, LANE), LANE)],
                    buf4.at[slot],
                    sem,
                )

            meta.append((pred, slot))
            fire_cnt = fire_cnt + pred.astype(jnp.int32)
            prev_c, prev_slot = c, slot
        return meta, (prev_c, fire_cnt, prev_slot)

    def select(g, jvec, i, meta):
        sem = sems[i % 2]
        for t in range(SUB):
            pred, slot = meta[t]

            @pl.when(pred)
            def _():
                pltpu.make_async_copy(
                    table_hbm.at[pl.ds(0, KTILE), pl.ds(0, 8), pl.ds(0, LANE)],
                    buf4.at[slot], sem,
                ).wait()

            e = g * 16 + SUB * i + t
            col = lax.broadcast(jvec[SUB * i + t], (16,))
            for kq in range(DIM // 16):
                rows = slot * DIM + kq * 16 + lane
                vals = plsc.load_gather(buf, [rows, col])
                slab[e, pl.ds(kq * 16, 16)] = vals

    def make_body(g0):
        def body(g, st):
            ids = idx_v[pl.ds(g * 16, 16)]
            cvec = ids >> 7
            jvec = ids & (LANE - 1)
            m0, st = fire(cvec, 0, st)
            m1, st = fire(cvec, 1, st)
            select(g - g0, jvec, 0, m0)
            m2, st = fire(cvec, 2, st)
            select(g - g0, jvec, 1, m1)
            m3, st = fire(cvec, 3, st)
            select(g - g0, jvec, 2, m2)
            select(g - g0, jvec, 3, m3)
            return st
        return body

    half_g = B_PER_W // 32
    st = (jnp.int32(-1), jnp.int32(0), jnp.int32(0))
    st = lax.fori_loop(0, half_g, make_body(0), st)
    pltpu.sync_copy(slab, out_hbm.at[pl.ds(base, B_PER_W // 2), pl.ds(0, DIM)])
    st = lax.fori_loop(half_g, 2 * half_g, make_body(half_g), st)
    pltpu.sync_copy(
        slab, out_hbm.at[pl.ds(base + B_PER_W // 2, B_PER_W // 2),
                         pl.ds(0, DIM)])


def kernel(embedding_weight, batch):
    table4 = embedding_weight.T.reshape(KTILE, 8, ROWS)
    idx = batch.astype(jnp.int32)
    ids_sorted, srt = lax.sort(
        [idx, jnp.arange(BATCH, dtype=jnp.int32)], num_keys=1)
    out2 = _sc_gather(table4, ids_sorted)
    inv = jnp.zeros((BATCH,), jnp.int32).at[srt].set(
        jnp.arange(BATCH, dtype=jnp.int32))
    return out2[inv]
